# baseline (speedup 1.0000x reference)
import functools

import jax
import jax.numpy as jnp
from jax import lax
from jax.experimental import pallas as pl
from jax.experimental.pallas import tpu as pltpu
from jax.experimental.pallas import tpu_sc as plsc

BATCH_SIZE = 4096


def _make_gather_kernel(n_rows, dx, dy, b):
    info = plsc.get_sparse_core_info()
    nc, ns = info.num_cores, info.num_subcores
    nw = nc * ns
    b_per_w = b // nw
    mesh = plsc.VectorSubcoreMesh(core_axis_name="c", subcore_axis_name="s")

    @functools.partial(
        pl.kernel,
        mesh=mesh,
        out_type=[
            jax.ShapeDtypeStruct((b, dx), jnp.float32),
            jax.ShapeDtypeStruct((b, dy), jnp.float32),
        ],
        scratch_types=[
            pltpu.VMEM((b_per_w,), jnp.int32),
            pltpu.VMEM((b_per_w, dx), jnp.float32),
            pltpu.VMEM((b_per_w, dy), jnp.float32),
            pltpu.SemaphoreType.DMA,
            pltpu.SemaphoreType.DMA,
        ],
    )
    def gather_kernel(x_hbm, y_hbm, idx_hbm, out_x_hbm, out_y_hbm,
                      idx_v, xrows, yrows, sem_x, sem_y):
        wid = lax.axis_index("s") * nc + lax.axis_index("c")
        base = wid * b_per_w
        pltpu.sync_copy(idx_hbm.at[pl.ds(base, b_per_w)], idx_v)

        pltpu.sync_copy(xrows, out_x_hbm.at[pl.ds(base, b_per_w)])
        pltpu.sync_copy(yrows, out_y_hbm.at[pl.ds(base, b_per_w)])

    return gather_kernel


def kernel(x_array, y_array, step):
    n = x_array.shape[0]
    dx, dy = x_array.shape[1], y_array.shape[1]
    num_batches = n // BATCH_SIZE
    epoch = step // num_batches
    k = jax.random.fold_in(jax.random.key(42), epoch)
    perm = jax.random.permutation(k, jnp.arange(n))
    start = (step % num_batches) * BATCH_SIZE
    batch_indices = lax.dynamic_slice_in_dim(perm, start, BATCH_SIZE)
    gather = _make_gather_kernel(n, dx, dy, BATCH_SIZE)
    out_x, out_y = gather(x_array.reshape(n // 8, 8, dx), y_array.reshape(n // 8, 8, dy), batch_indices.astype(jnp.int32))
    return (out_x, out_y)


# speedup vs baseline: 1.8263x; 1.8263x over previous
import functools

import jax
import jax.numpy as jnp
from jax import lax
from jax.experimental import pallas as pl
from jax.experimental.pallas import tpu as pltpu
from jax.experimental.pallas import tpu_sc as plsc

B = 4096
N = 1000000
NB = 4096
NPAD = 1000448
PERW = NPAD // 32
SLAB = 384
MCAP = 384
SPAN = 64

_params = pltpu.CompilerParams(needs_layout_passes=False)


def _mesh():
    return plsc.VectorSubcoreMesh(core_axis_name="c", subcore_axis_name="s")


def _wid():
    info = plsc.get_sparse_core_info()
    return lax.axis_index("s") * info.num_cores + lax.axis_index("c")


def _splat(ref2d, row, col):
    r = jnp.full((16,), row, dtype=jnp.int32)
    c = jnp.full((16,), col, dtype=jnp.int32)
    return plsc.load_gather(ref2d, [r, c])


@functools.partial(
    pl.kernel, mesh=_mesh(),
    out_type=jax.ShapeDtypeStruct((32, 16, NB), jnp.int32),
    scratch_types=[
        pltpu.VMEM((PERW,), jnp.int32),
        pltpu.VMEM((1, 16, NB), jnp.int32),
    ],
    compiler_params=_params,
)
def _k1_hist(keys_hbm, hist_hbm, keys_v, hist_v):
    w = _wid()
    pltpu.sync_copy(keys_hbm.at[pl.ds(w * PERW, PERW)], keys_v)
    zero = jnp.zeros((16,), jnp.int32)

    def zinit(k):
        for l in range(16):
            hist_v[0, l, pl.ds(k * 16, 16)] = zero

    pl.loop(0, NB // 16)(zinit)

    lanes = lax.iota(jnp.int32, 16)
    ones = jnp.ones((16,), jnp.int32)
    zeros16 = jnp.zeros((16,), jnp.int32)

    def body(i):
        kv = keys_v[pl.ds(i * 16, 16)]
        bv = lax.shift_right_logical(kv, 20)
        plsc.addupdate_scatter(hist_v, [zeros16, lanes, bv], ones)

    pl.loop(0, PERW // 16)(body)
    pltpu.sync_copy(hist_v, hist_hbm.at[pl.ds(w, 1)])


@functools.partial(
    pl.kernel, mesh=_mesh(),
    out_type=[
        jax.ShapeDtypeStruct((32, 128), jnp.int32),
        jax.ShapeDtypeStruct((32, 16), jnp.int32),
    ],
    scratch_types=[
        pltpu.VMEM((32, 16, 128), jnp.int32),
        pltpu.VMEM((1, 128), jnp.int32),
        pltpu.VMEM((1, 16), jnp.int32),
    ],
    compiler_params=_params,
)
def _k2_sum(hist_hbm, lp_hbm, tot_hbm, buf, lp_v, tot_v):
    w = _wid()
    pltpu.sync_copy(hist_hbm.at[:, :, pl.ds(w * 128, 128)], buf)

    accs = tuple(jnp.zeros((16,), jnp.int32) for _ in range(8))

    def body(rr, carry):
        ww = rr >> 4
        l = rr & 15
        return tuple(carry[v] + buf[ww, l, pl.ds(v * 16, 16)]
                     for v in range(8))

    accs = pl.loop(0, 512, init_carry=accs)(body)

    carry = jnp.zeros((16,), jnp.int32)
    for v in range(8):
        inc = plsc.cumsum(accs[v])
        excl = inc - accs[v] + carry
        lp_v[0, pl.ds(v * 16, 16)] = excl
        carry = jnp.full((16,), inc[15], jnp.int32) + carry
    tot_v[0, :] = carry
    pltpu.sync_copy(lp_v, lp_hbm.at[pl.ds(w, 1)])
    pltpu.sync_copy(tot_v, tot_hbm.at[pl.ds(w, 1)])


@functools.partial(
    pl.kernel, mesh=_mesh(),
    out_type=[
        jax.ShapeDtypeStruct((16,), jnp.int32),
        jax.ShapeDtypeStruct((SPAN,), jnp.int32),
    ],
    scratch_types=[
        pltpu.VMEM((32, 16), jnp.int32),
        pltpu.VMEM((32, 128), jnp.int32),
        pltpu.VMEM((16,), jnp.int32),
        pltpu.VMEM((32,), jnp.int32),
        pltpu.VMEM((NB + SPAN,), jnp.int32),
        pltpu.VMEM((16,), jnp.int32),
        pltpu.VMEM((SPAN,), jnp.int32),
    ],
    compiler_params=_params,
)
def _k3_meta(tot_hbm, lp_hbm, start_hbm, meta_hbm, cbase_hbm,
             tot_v, lp_v, start_v, base_v, c_v, meta_v, cb_v):
    w = _wid()

    @pl.when(w == 0)
    def _():
        pltpu.sync_copy(tot_hbm, tot_v)
        pltpu.sync_copy(lp_hbm, lp_v)
        pltpu.sync_copy(start_hbm, start_v)
        lanes = lax.iota(jnp.int32, 16)
        zeros16 = jnp.zeros((16,), jnp.int32)

        t0 = plsc.load_gather(tot_v, [lanes, lanes])
        t1 = plsc.load_gather(tot_v, [lanes + 16, lanes])
        c0 = plsc.cumsum(t0)
        c1 = plsc.cumsum(t1)
        base_v[pl.ds(0, 16)] = c0 - t0
        base_v[pl.ds(16, 16)] = c1 - t1 + jnp.full((16,), c0[15], jnp.int32)

        start_splat = start_v[pl.ds(0, 16)]
        end_splat = start_splat + (B - 1)

        def scan(i, carry):
            acc_lo, acc_hi = carry
            lpv = lp_v[i >> 3, pl.ds((i & 7) * 16, 16)]
            bidx = (jnp.full((16,), i * 16, jnp.int32) + lanes) >> 7
            cv = plsc.load_gather(base_v, [bidx]) + lpv
            c_v[pl.ds(i * 16, 16)] = cv
            acc_lo = acc_lo + jnp.where(cv <= start_splat, 1, 0)
            acc_hi = acc_hi + jnp.where(cv <= end_splat, 1, 0)
            return acc_lo, acc_hi

        acc_lo, acc_hi = pl.loop(
            0, NB // 16,
            init_carry=(jnp.zeros((16,), jnp.int32),
                        jnp.zeros((16,), jnp.int32)))(scan)
        b_lo = jnp.sum(acc_lo) - 1
        b_hi = jnp.sum(acc_hi) - 1
        meta_v[...] = jnp.where(lanes == 0, b_lo,
                                jnp.where(lanes == 1, b_hi, 0))
        pltpu.sync_copy(meta_v, meta_hbm)
        for k in range(SPAN // 16):
            cb_v[pl.ds(k * 16, 16)] = c_v[pl.ds(b_lo + k * 16, 16)]
        pltpu.sync_copy(cb_v, cbase_hbm)


@functools.partial(
    pl.kernel, mesh=_mesh(),
    out_type=[
        jax.ShapeDtypeStruct((32, SLAB), jnp.int32),
        jax.ShapeDtypeStruct((32, SLAB), jnp.int32),
        jax.ShapeDtypeStruct((32, 16), jnp.int32),
    ],
    scratch_types=[
        pltpu.VMEM((PERW,), jnp.int32),
        pltpu.VMEM((16,), jnp.int32),
        pltpu.VMEM((1, SLAB + 16), jnp.int32),
        pltpu.VMEM((1, SLAB + 16), jnp.int32),
        pltpu.VMEM((1, 16), jnp.int32),
    ],
    compiler_params=_params,
)
def _k4_collect(keys_hbm, meta_hbm, sk_hbm, sp_hbm, cnt_hbm,
                keys_v, meta_v, sk_v, sp_v, cnt_v):
    w = _wid()
    pltpu.sync_copy(keys_hbm.at[pl.ds(w * PERW, PERW)], keys_v)
    pltpu.sync_copy(meta_hbm, meta_v)
    lanes = lax.iota(jnp.int32, 16)
    meta_vec = meta_v[pl.ds(0, 16)]
    b_lo = jnp.full((16,), meta_vec[0], jnp.int32)
    b_hi = jnp.full((16,), meta_vec[1], jnp.int32)
    pos0 = jnp.full((16,), w * PERW, jnp.int32) + lanes

    zl = jnp.zeros((16,), jnp.int32)

    def body(i, cur):
        kv = keys_v[pl.ds(i * 16, 16)]
        bv = lax.shift_right_logical(kv, 20)
        mask = (bv >= b_lo) & (bv <= b_hi)
        mi = jnp.where(mask, 1, 0)
        inc = plsc.cumsum(mi)
        dest = jnp.full((16,), cur, jnp.int32) + inc - mi
        plsc.store_scatter(sk_v, [zl, dest], kv, mask=mask)
        plsc.store_scatter(sp_v, [zl, dest], pos0 + i * 16, mask=mask)
        return cur + inc[15]

    cur = pl.loop(0, PERW // 16, init_carry=jnp.int32(0))(body)
    cnt_v[0, :] = jnp.full((16,), cur, jnp.int32)
    pltpu.sync_copy(sk_v.at[:, pl.ds(0, SLAB)], sk_hbm.at[pl.ds(w, 1)])
    pltpu.sync_copy(sp_v.at[:, pl.ds(0, SLAB)], sp_hbm.at[pl.ds(w, 1)])
    pltpu.sync_copy(cnt_v, cnt_hbm.at[pl.ds(w, 1)])


@functools.partial(
    pl.kernel, mesh=_mesh(),
    out_type=jax.ShapeDtypeStruct((B, 16), jnp.int32),
    scratch_types=[
        pltpu.VMEM((32, SLAB), jnp.int32),
        pltpu.VMEM((32, SLAB), jnp.int32),
        pltpu.VMEM((32, 16), jnp.int32),
        pltpu.VMEM((16,), jnp.int32),
        pltpu.VMEM((SPAN,), jnp.int32),
        pltpu.VMEM((16,), jnp.int32),
        pltpu.VMEM((MCAP + 16,), jnp.int32),
        pltpu.VMEM((MCAP + 16,), jnp.int32),
        pltpu.VMEM((1, 16), jnp.int32),
    ],
    compiler_params=_params,
)
def _k5_select(sk_hbm, sp_hbm, cnt_hbm, meta_hbm, cbase_hbm, start_hbm,
               j_hbm, sk_v, sp_v, cnt_v, meta_v, cb_v, start_v,
               mk_v, mp_v, tmp_v):
    w = _wid()
    pltpu.sync_copy(sk_hbm, sk_v)
    pltpu.sync_copy(sp_hbm, sp_v)
    pltpu.sync_copy(cnt_hbm, cnt_v)
    pltpu.sync_copy(meta_hbm, meta_v)
    pltpu.sync_copy(cbase_hbm, cb_v)
    pltpu.sync_copy(start_hbm, start_v)
    lanes = lax.iota(jnp.int32, 16)
    meta_vec = meta_v[pl.ds(0, 16)]
    b_lo = jnp.full((16,), meta_vec[0], jnp.int32)
    b_hi = jnp.full((16,), meta_vec[1], jnp.int32)
    start_splat = start_v[pl.ds(0, 16)]

    for rnd in range(SPAN // 32):
        boff = rnd * 32 + w
        b_splat = b_lo + boff
        bucket = b_splat[0]

        @pl.when(bucket <= b_hi[0])
        def _(boff=boff, b_splat=b_splat):

            def collect(s, cur):
                cnt_s = cnt_v[s, pl.ds(0, 16)]
                got = jnp.int32(0)
                for v in range(SLAB // 16):
                    kv = sk_v[s, pl.ds(v * 16, 16)]
                    bv = lax.shift_right_logical(kv, 20)
                    mask = (bv == b_splat) & (lanes + v * 16 < cnt_s)
                    mi = jnp.where(mask, 1, 0)
                    inc = plsc.cumsum(mi)
                    dest = jnp.full((16,), cur + got, jnp.int32) + inc - mi
                    plsc.store_scatter(mk_v, [dest], kv, mask=mask)
                    plsc.store_scatter(mp_v, [dest],
                                       sp_v[s, pl.ds(v * 16, 16)], mask=mask)
                    got = got + inc[15]
                return cur + got

            m = pl.loop(0, 32, init_carry=jnp.int32(0))(collect)
            cbase = plsc.load_gather(
                cb_v, [jnp.full((16,), boff, jnp.int32)])
            nvec = (m + 15) >> 4

            def rank_one(j, m=m, cbase=cbase, nvec=nvec):
                js = jnp.full((16,), j, jnp.int32)
                kj = plsc.load_gather(mk_v, [js])
                pj = plsc.load_gather(mp_v, [js])

                def cmp(v, acc, kj=kj, pj=pj, m=m):
                    sb = jnp.int32(-2147483648)
                    kv = mk_v[pl.ds(v * 16, 16)] ^ sb
                    pv = mp_v[pl.ds(v * 16, 16)]
                    valid = lanes + v * 16 < m
                    kjx = kj ^ sb
                    less = (kv < kjx) | ((kv == kjx) & (pv < pj))
                    return acc + jnp.where(less & valid, 1, 0)

                acc = pl.loop(0, nvec,
                              init_carry=jnp.zeros((16,), jnp.int32))(cmp)
                rank = jnp.sum(acc) + cbase[0]
                t = rank - start_splat[0]

                @pl.when((t >= 0) & (t < B))
                def _(pj=pj, t=t):
                    tmp_v[0, :] = pj
                    pltpu.sync_copy(tmp_v, j_hbm.at[pl.ds(t, 1)])

            pl.loop(0, m)(rank_one)


def _make_k6(dx, dy):
    b_per_w = B // 32

    @functools.partial(
        pl.kernel, mesh=_mesh(),
        out_type=[
            jax.ShapeDtypeStruct((B, 1, dx), jnp.float32),
            jax.ShapeDtypeStruct((B, 1, dy), jnp.float32),
        ],
        scratch_types=[
            pltpu.VMEM((b_per_w, 16), jnp.int32),
            pltpu.VMEM((b_per_w, 16), jnp.int32),
            pltpu.VMEM((b_per_w, 1, dx), jnp.float32),
            pltpu.VMEM((b_per_w, 1, dy), jnp.float32),
            pltpu.SemaphoreType.DMA,
            pltpu.SemaphoreType.DMA,
            pltpu.SemaphoreType.DMA,
        ],
        compiler_params=_params,
    )
    def k6(j_hbm, x1_hbm, x_hbm, y_hbm, ox_hbm, oy_hbm,
           j_v, bi_v, xrows, yrows, sem_b, sem_x, sem_y):
        w = _wid()
        base = w * b_per_w
        pltpu.sync_copy(j_hbm.at[pl.ds(base, b_per_w)], j_v)
        lanes = lax.iota(jnp.int32, 16)
        zeros16 = jnp.zeros((16,), jnp.int32)

        def issue_b(kb):
            jv = plsc.load_gather(j_v, [lanes + kb * 16, lanes])
            jhi = jv >> 4
            for l in range(16):
                pltpu.async_copy(x1_hbm.at[pl.ds(jhi[l], 1)],
                                 bi_v.at[pl.ds(kb * 16 + l, 1)], sem_b)

        pl.loop(0, b_per_w // 16)(issue_b)

        def drain_b(j):
            pltpu.make_async_copy(x1_hbm.at[pl.ds(0, 1)],
                                  bi_v.at[pl.ds(j, 1)], sem_b).wait()

        pl.loop(0, b_per_w)(drain_b)

        def issue_rows(kb):
            jv = plsc.load_gather(j_v, [lanes + kb * 16, lanes])
            jlo = jv & 15
            bi = plsc.load_gather(bi_v, [lanes + kb * 16, jlo])
            bhi = bi >> 3
            blo = bi & 7
            for l in range(16):
                j = kb * 16 + l
                pltpu.async_copy(
                    x_hbm.at[pl.ds(bhi[l], 1), pl.ds(blo[l], 1)],
                    xrows.at[pl.ds(j, 1)], sem_x)
                pltpu.async_copy(
                    y_hbm.at[pl.ds(bhi[l], 1), pl.ds(blo[l], 1)],
                    yrows.at[pl.ds(j, 1)], sem_y)

        pl.loop(0, b_per_w // 16)(issue_rows)

        def drain_rows(j):
            pltpu.make_async_copy(
                x_hbm.at[pl.ds(0, 1), pl.ds(0, 1)],
                xrows.at[pl.ds(j, 1)], sem_x).wait()
            pltpu.make_async_copy(
                y_hbm.at[pl.ds(0, 1), pl.ds(0, 1)],
                yrows.at[pl.ds(j, 1)], sem_y).wait()

        pl.loop(0, b_per_w)(drain_rows)

        pltpu.sync_copy(xrows, ox_hbm.at[pl.ds(base, b_per_w)])
        pltpu.sync_copy(yrows, oy_hbm.at[pl.ds(base, b_per_w)])

    return k6


def kernel(x_array, y_array, step):
    n = x_array.shape[0]
    dx, dy = x_array.shape[1], y_array.shape[1]
    num_batches = n // B
    epoch = step // num_batches
    key = jax.random.fold_in(jax.random.key(42), epoch)
    key, sub1 = jax.random.split(key)
    keys1 = jax.random.bits(sub1, (n,), jnp.uint32)
    _, x1 = lax.sort_key_val(keys1, jnp.arange(n, dtype=jnp.int32))
    key, sub2 = jax.random.split(key)
    keys2 = jax.random.bits(sub2, (n,), jnp.uint32)
    start = (step % num_batches) * B

    keys2x = lax.bitcast_convert_type(keys2, jnp.int32)
    keys2p = jnp.concatenate(
        [keys2x, jnp.full((NPAD - n,), -1, jnp.int32)])
    start_arr = jnp.full((16,), start, jnp.int32)

    hist = _k1_hist(keys2p)
    lp, tot = _k2_sum(hist)
    meta, cbase = _k3_meta(tot, lp, start_arr)
    sk, sp, cnt = _k4_collect(keys2p, meta)
    j_arr = _k5_select(sk, sp, cnt, meta, cbase, start_arr)
    k6 = _make_k6(dx, dy)
    out_x, out_y = k6(
        j_arr, x1.reshape(n // 16, 16), x_array.reshape(n // 8, 8, dx),
        y_array.reshape(n // 8, 8, dy))
    return (out_x.reshape(B, dx), out_y.reshape(B, dy))


# speedup vs baseline: 1.8268x; 1.0003x over previous
import functools

import jax
import jax.numpy as jnp
from jax import lax
from jax.experimental import pallas as pl
from jax.experimental.pallas import tpu as pltpu
from jax.experimental.pallas import tpu_sc as plsc

B = 4096
N = 1000000
NB = 4096
NPAD = 1000448
PERW = NPAD // 32
SLAB = 384
MCAP = 384
SPAN = 64

_params = pltpu.CompilerParams(needs_layout_passes=False)


def _mesh():
    return plsc.VectorSubcoreMesh(core_axis_name="c", subcore_axis_name="s")


def _wid():
    info = plsc.get_sparse_core_info()
    return lax.axis_index("s") * info.num_cores + lax.axis_index("c")


@functools.partial(
    pl.kernel, mesh=_mesh(),
    out_type=jax.ShapeDtypeStruct((32, 16, NB), jnp.int32),
    scratch_types=[
        pltpu.VMEM((PERW,), jnp.int32),
        pltpu.VMEM((1, 16, NB), jnp.int32),
    ],
    compiler_params=_params,
)
def _k1_hist(keys_hbm, hist_hbm, keys_v, hist_v):
    w = _wid()
    pltpu.sync_copy(keys_hbm.at[pl.ds(w * PERW, PERW)], keys_v)
    zero = jnp.zeros((16,), jnp.int32)

    def zinit(k):
        for l in range(16):
            hist_v[0, l, pl.ds(k * 16, 16)] = zero

    pl.loop(0, NB // 16)(zinit)

    lanes = lax.iota(jnp.int32, 16)
    ones = jnp.ones((16,), jnp.int32)
    zeros16 = jnp.zeros((16,), jnp.int32)

    def body(i):
        kv = keys_v[pl.ds(i * 16, 16)]
        bv = lax.shift_right_logical(kv, 20)
        plsc.addupdate_scatter(hist_v, [zeros16, lanes, bv], ones)

    pl.loop(0, PERW // 16)(body)
    pltpu.sync_copy(hist_v, hist_hbm.at[pl.ds(w, 1)])


@functools.partial(
    pl.kernel, mesh=_mesh(),
    out_type=[
        jax.ShapeDtypeStruct((32, 128), jnp.int32),
        jax.ShapeDtypeStruct((32, 16), jnp.int32),
    ],
    scratch_types=[
        pltpu.VMEM((32, 16, 128), jnp.int32),
        pltpu.VMEM((1, 128), jnp.int32),
        pltpu.VMEM((1, 16), jnp.int32),
    ],
    compiler_params=_params,
)
def _k2_sum(hist_hbm, lp_hbm, tot_hbm, buf, lp_v, tot_v):
    w = _wid()
    pltpu.sync_copy(hist_hbm.at[:, :, pl.ds(w * 128, 128)], buf)

    accs = tuple(jnp.zeros((16,), jnp.int32) for _ in range(8))

    def body(rr, carry):
        ww = rr >> 4
        l = rr & 15
        return tuple(carry[v] + buf[ww, l, pl.ds(v * 16, 16)]
                     for v in range(8))

    accs = pl.loop(0, 512, init_carry=accs)(body)

    carry = jnp.zeros((16,), jnp.int32)
    for v in range(8):
        inc = plsc.cumsum(accs[v])
        excl = inc - accs[v] + carry
        lp_v[0, pl.ds(v * 16, 16)] = excl
        carry = jnp.full((16,), inc[15], jnp.int32) + carry
    tot_v[0, :] = carry
    pltpu.sync_copy(lp_v, lp_hbm.at[pl.ds(w, 1)])
    pltpu.sync_copy(tot_v, tot_hbm.at[pl.ds(w, 1)])


@functools.partial(
    pl.kernel, mesh=_mesh(),
    out_type=[
        jax.ShapeDtypeStruct((16,), jnp.int32),
        jax.ShapeDtypeStruct((SPAN,), jnp.int32),
    ],
    scratch_types=[
        pltpu.VMEM((32, 16), jnp.int32),
        pltpu.VMEM((32, 128), jnp.int32),
        pltpu.VMEM((16,), jnp.int32),
        pltpu.VMEM((32,), jnp.int32),
        pltpu.VMEM((NB + SPAN,), jnp.int32),
        pltpu.VMEM((16,), jnp.int32),
        pltpu.VMEM((SPAN,), jnp.int32),
    ],
    compiler_params=_params,
)
def _k3_meta(tot_hbm, lp_hbm, start_hbm, meta_hbm, cbase_hbm,
             tot_v, lp_v, start_v, base_v, c_v, meta_v, cb_v):
    w = _wid()

    @pl.when(w == 0)
    def _():
        pltpu.sync_copy(tot_hbm, tot_v)
        pltpu.sync_copy(lp_hbm, lp_v)
        pltpu.sync_copy(start_hbm, start_v)
        lanes = lax.iota(jnp.int32, 16)
        zeros16 = jnp.zeros((16,), jnp.int32)

        t0 = plsc.load_gather(tot_v, [lanes, lanes])
        t1 = plsc.load_gather(tot_v, [lanes + 16, lanes])
        c0 = plsc.cumsum(t0)
        c1 = plsc.cumsum(t1)
        base_v[pl.ds(0, 16)] = c0 - t0
        base_v[pl.ds(16, 16)] = c1 - t1 + jnp.full((16,), c0[15], jnp.int32)

        start_splat = start_v[pl.ds(0, 16)]
        end_splat = start_splat + (B - 1)

        def scan(i, carry):
            acc_lo, acc_hi = carry
            lpv = lp_v[i >> 3, pl.ds((i & 7) * 16, 16)]
            bidx = (jnp.full((16,), i * 16, jnp.int32) + lanes) >> 7
            cv = plsc.load_gather(base_v, [bidx]) + lpv
            c_v[pl.ds(i * 16, 16)] = cv
            acc_lo = acc_lo + jnp.where(cv <= start_splat, 1, 0)
            acc_hi = acc_hi + jnp.where(cv <= end_splat, 1, 0)
            return acc_lo, acc_hi

        acc_lo, acc_hi = pl.loop(
            0, NB // 16,
            init_carry=(jnp.zeros((16,), jnp.int32),
                        jnp.zeros((16,), jnp.int32)))(scan)
        b_lo = jnp.sum(acc_lo) - 1
        b_hi = jnp.sum(acc_hi) - 1
        meta_v[...] = jnp.where(lanes == 0, b_lo,
                                jnp.where(lanes == 1, b_hi, 0))
        pltpu.sync_copy(meta_v, meta_hbm)
        for k in range(SPAN // 16):
            cb_v[pl.ds(k * 16, 16)] = c_v[pl.ds(b_lo + k * 16, 16)]
        pltpu.sync_copy(cb_v, cbase_hbm)


@functools.partial(
    pl.kernel, mesh=_mesh(),
    out_type=[
        jax.ShapeDtypeStruct((32, SLAB), jnp.int32),
        jax.ShapeDtypeStruct((32, SLAB), jnp.int32),
        jax.ShapeDtypeStruct((32, 16), jnp.int32),
    ],
    scratch_types=[
        pltpu.VMEM((PERW,), jnp.int32),
        pltpu.VMEM((16,), jnp.int32),
        pltpu.VMEM((1, SLAB + 16), jnp.int32),
        pltpu.VMEM((1, SLAB + 16), jnp.int32),
        pltpu.VMEM((1, 16), jnp.int32),
    ],
    compiler_params=_params,
)
def _k4_collect(keys_hbm, meta_hbm, sk_hbm, sp_hbm, cnt_hbm,
                keys_v, meta_v, sk_v, sp_v, cnt_v):
    w = _wid()
    pltpu.sync_copy(keys_hbm.at[pl.ds(w * PERW, PERW)], keys_v)
    pltpu.sync_copy(meta_hbm, meta_v)
    lanes = lax.iota(jnp.int32, 16)
    meta_vec = meta_v[pl.ds(0, 16)]
    b_lo = jnp.full((16,), meta_vec[0], jnp.int32)
    b_hi = jnp.full((16,), meta_vec[1], jnp.int32)
    pos0 = jnp.full((16,), w * PERW, jnp.int32) + lanes

    zl = jnp.zeros((16,), jnp.int32)

    def body(i, cur):
        kv = keys_v[pl.ds(i * 16, 16)]
        bv = lax.shift_right_logical(kv, 20)
        mask = (bv >= b_lo) & (bv <= b_hi)
        mi = jnp.where(mask, 1, 0)
        inc = plsc.cumsum(mi)
        dest = jnp.full((16,), cur, jnp.int32) + inc - mi
        plsc.store_scatter(sk_v, [zl, dest], kv, mask=mask)
        plsc.store_scatter(sp_v, [zl, dest], pos0 + i * 16, mask=mask)
        return cur + inc[15]

    cur = pl.loop(0, PERW // 16, init_carry=jnp.int32(0))(body)
    cnt_v[0, :] = jnp.full((16,), cur, jnp.int32)
    pltpu.sync_copy(sk_v.at[:, pl.ds(0, SLAB)], sk_hbm.at[pl.ds(w, 1)])
    pltpu.sync_copy(sp_v.at[:, pl.ds(0, SLAB)], sp_hbm.at[pl.ds(w, 1)])
    pltpu.sync_copy(cnt_v, cnt_hbm.at[pl.ds(w, 1)])


@functools.partial(
    pl.kernel, mesh=_mesh(),
    out_type=jax.ShapeDtypeStruct((B, 16), jnp.int32),
    scratch_types=[
        pltpu.VMEM((32, SLAB), jnp.int32),
        pltpu.VMEM((32, SLAB), jnp.int32),
        pltpu.VMEM((32, 16), jnp.int32),
        pltpu.VMEM((16,), jnp.int32),
        pltpu.VMEM((SPAN,), jnp.int32),
        pltpu.VMEM((16,), jnp.int32),
        pltpu.VMEM((MCAP + 16,), jnp.int32),
        pltpu.VMEM((MCAP + 16,), jnp.int32),
        pltpu.VMEM((1, 16), jnp.int32),
    ],
    compiler_params=_params,
)
def _k5_select(sk_hbm, sp_hbm, cnt_hbm, meta_hbm, cbase_hbm, start_hbm,
               j_hbm, sk_v, sp_v, cnt_v, meta_v, cb_v, start_v,
               mk_v, mp_v, tmp_v):
    w = _wid()
    pltpu.sync_copy(sk_hbm, sk_v)
    pltpu.sync_copy(sp_hbm, sp_v)
    pltpu.sync_copy(cnt_hbm, cnt_v)
    pltpu.sync_copy(meta_hbm, meta_v)
    pltpu.sync_copy(cbase_hbm, cb_v)
    pltpu.sync_copy(start_hbm, start_v)
    lanes = lax.iota(jnp.int32, 16)
    meta_vec = meta_v[pl.ds(0, 16)]
    b_lo = jnp.full((16,), meta_vec[0], jnp.int32)
    b_hi = jnp.full((16,), meta_vec[1], jnp.int32)
    start_splat = start_v[pl.ds(0, 16)]

    for rnd in range(SPAN // 32):
        boff = rnd * 32 + w
        b_splat = b_lo + boff
        bucket = b_splat[0]

        @pl.when(bucket <= b_hi[0])
        def _(boff=boff, b_splat=b_splat):

            def collect(s, cur):
                cnt_s = cnt_v[s, pl.ds(0, 16)]
                got = jnp.int32(0)
                for v in range(SLAB // 16):
                    kv = sk_v[s, pl.ds(v * 16, 16)]
                    bv = lax.shift_right_logical(kv, 20)
                    mask = (bv == b_splat) & (lanes + v * 16 < cnt_s)
                    mi = jnp.where(mask, 1, 0)
                    inc = plsc.cumsum(mi)
                    dest = jnp.full((16,), cur + got, jnp.int32) + inc - mi
                    plsc.store_scatter(mk_v, [dest], kv, mask=mask)
                    plsc.store_scatter(mp_v, [dest],
                                       sp_v[s, pl.ds(v * 16, 16)], mask=mask)
                    got = got + inc[15]
                return cur + got

            m = pl.loop(0, 32, init_carry=jnp.int32(0))(collect)
            cbase = plsc.load_gather(
                cb_v, [jnp.full((16,), boff, jnp.int32)])
            nvec = (m + 15) >> 4

            def rank_one(j, m=m, cbase=cbase, nvec=nvec):
                js = jnp.full((16,), j, jnp.int32)
                kj = plsc.load_gather(mk_v, [js])
                pj = plsc.load_gather(mp_v, [js])

                def cmp(v, acc, kj=kj, pj=pj, m=m):
                    sb = jnp.int32(-2147483648)
                    kv = mk_v[pl.ds(v * 16, 16)] ^ sb
                    pv = mp_v[pl.ds(v * 16, 16)]
                    valid = lanes + v * 16 < m
                    kjx = kj ^ sb
                    less = (kv < kjx) | ((kv == kjx) & (pv < pj))
                    return acc + jnp.where(less & valid, 1, 0)

                acc = pl.loop(0, nvec,
                              init_carry=jnp.zeros((16,), jnp.int32))(cmp)
                rank = jnp.sum(acc) + cbase[0]
                t = rank - start_splat[0]

                @pl.when((t >= 0) & (t < B))
                def _(pj=pj, t=t):
                    tmp_v[0, :] = pj
                    pltpu.sync_copy(tmp_v, j_hbm.at[pl.ds(t, 1)])

            pl.loop(0, m)(rank_one)


def _make_k6(dx, dy):
    b_per_w = B // 32

    @functools.partial(
        pl.kernel, mesh=_mesh(),
        out_type=[
            jax.ShapeDtypeStruct((B, 1, dx), jnp.float32),
            jax.ShapeDtypeStruct((B, 1, dy), jnp.float32),
        ],
        scratch_types=[
            pltpu.VMEM((b_per_w, 16), jnp.int32),
            pltpu.VMEM((b_per_w, 16), jnp.int32),
            pltpu.VMEM((b_per_w, 1, dx), jnp.float32),
            pltpu.VMEM((b_per_w, 1, dy), jnp.float32),
            pltpu.SemaphoreType.DMA,
            pltpu.SemaphoreType.DMA,
            pltpu.SemaphoreType.DMA,
        ],
        compiler_params=_params,
    )
    def k6(j_hbm, x1_hbm, x_hbm, y_hbm, ox_hbm, oy_hbm,
           j_v, bi_v, xrows, yrows, sem_b, sem_x, sem_y):
        w = _wid()
        base = w * b_per_w
        pltpu.sync_copy(j_hbm.at[pl.ds(base, b_per_w)], j_v)
        lanes = lax.iota(jnp.int32, 16)
        zeros16 = jnp.zeros((16,), jnp.int32)

        def issue_b(kb):
            jv = plsc.load_gather(j_v, [lanes + kb * 16, lanes])
            jhi = jv >> 4
            for l in range(16):
                pltpu.async_copy(x1_hbm.at[pl.ds(jhi[l], 1)],
                                 bi_v.at[pl.ds(kb * 16 + l, 1)], sem_b)

        pl.loop(0, b_per_w // 16)(issue_b)

        def drain_b(j):
            pltpu.make_async_copy(x1_hbm.at[pl.ds(0, 1)],
                                  bi_v.at[pl.ds(j, 1)], sem_b).wait()

        pl.loop(0, b_per_w)(drain_b)

        def issue_rows(kb):
            jv = plsc.load_gather(j_v, [lanes + kb * 16, lanes])
            jlo = jv & 15
            bi = plsc.load_gather(bi_v, [lanes + kb * 16, jlo])
            bhi = bi >> 3
            blo = bi & 7
            for l in range(16):
                j = kb * 16 + l
                pltpu.async_copy(
                    x_hbm.at[pl.ds(bhi[l], 1), pl.ds(blo[l], 1)],
                    xrows.at[pl.ds(j, 1)], sem_x)
                pltpu.async_copy(
                    y_hbm.at[pl.ds(bhi[l], 1), pl.ds(blo[l], 1)],
                    yrows.at[pl.ds(j, 1)], sem_y)

        pl.loop(0, b_per_w // 16)(issue_rows)

        def drain_rows(j):
            pltpu.make_async_copy(
                x_hbm.at[pl.ds(0, 1), pl.ds(0, 1)],
                xrows.at[pl.ds(j, 1)], sem_x).wait()
            pltpu.make_async_copy(
                y_hbm.at[pl.ds(0, 1), pl.ds(0, 1)],
                yrows.at[pl.ds(j, 1)], sem_y).wait()

        pl.loop(0, b_per_w)(drain_rows)

        pltpu.sync_copy(xrows, ox_hbm.at[pl.ds(base, b_per_w)])
        pltpu.sync_copy(yrows, oy_hbm.at[pl.ds(base, b_per_w)])

    return k6


def kernel(x_array, y_array, step):
    n = x_array.shape[0]
    dx, dy = x_array.shape[1], y_array.shape[1]
    num_batches = n // B
    epoch = step // num_batches
    key = jax.random.fold_in(jax.random.key(42), epoch)
    key, sub1 = jax.random.split(key)
    keys1 = jax.random.bits(sub1, (n,), jnp.uint32)
    _, x1 = lax.sort_key_val(keys1, jnp.arange(n, dtype=jnp.int32))
    key, sub2 = jax.random.split(key)
    keys2 = jax.random.bits(sub2, (n,), jnp.uint32)
    start = (step % num_batches) * B

    keys2x = lax.bitcast_convert_type(keys2, jnp.int32)
    keys2p = jnp.concatenate(
        [keys2x, jnp.full((NPAD - n,), -1, jnp.int32)])
    start_arr = jnp.full((16,), start, jnp.int32)

    hist = _k1_hist(keys2p)
    lp, tot = _k2_sum(hist)
    meta, cbase = _k3_meta(tot, lp, start_arr)
    sk, sp, cnt = _k4_collect(keys2p, meta)
    j_arr = _k5_select(sk, sp, cnt, meta, cbase, start_arr)
    k6 = _make_k6(dx, dy)
    out_x, out_y = k6(
        j_arr, x1.reshape(n // 16, 16), x_array.reshape(n // 8, 8, dx),
        y_array.reshape(n // 8, 8, dy))
    return (out_x.reshape(B, dx), out_y.reshape(B, dy))
